# baseline (speedup 1.0000x reference)
import jax
import jax.numpy as jnp
from jax import lax
from jax.experimental import pallas as pl
from jax.experimental.pallas import tpu as pltpu
from jax.experimental.pallas import tpu_sc as plsc

_P = 8
_MASK_RATIO = 0.75
_CC = 64
_ROWS, _LANES = 224, 224
_NP = 784
_JPW = 112
_LN = 16


_GDN = lax.GatherDimensionNumbers(
    offset_dims=(), collapsed_slice_dims=(0,), start_index_map=(0,))


def _vbcast(v, l):
    idx = jnp.full((_LN, 1), l, jnp.int32)
    return lax.gather(v, idx, _GDN, slice_sizes=(1,),
                      mode=lax.GatherScatterMode.PROMISE_IN_BOUNDS)


def _sc_mask_kernel(noise_hbm, mask_hbm, noise_v, mask_v):
    b_total = noise_hbm.shape[0] // _NP
    num_keep = int(_NP * (1.0 - _MASK_RATIO))
    nworkers = b_total * (_NP // _JPW)

    wid = lax.axis_index("s") * 2 + lax.axis_index("c")

    @pl.when(wid < nworkers)
    def _():
        b = wid // (_NP // _JPW)
        part = wid % (_NP // _JPW)
        pltpu.sync_copy(noise_hbm.at[pl.ds(b * _NP, _NP)],
                        noise_v.at[pl.ds(0, _NP)])

        lane = lax.broadcasted_iota(jnp.int32, (_LN,), 0)
        one = jnp.ones((_LN,), jnp.float32)
        zero = jnp.zeros((_LN,), jnp.float32)
        nkc = _NP // _LN

        for jc in range(_JPW // _LN):
            j0 = part * _JPW + jc * _LN
            njv = noise_v[pl.ds(j0, _LN)]
            jidx = lane + j0
            kb = part * (_JPW // _LN) + jc

            def before(kc, cnt):
                vk = noise_v[pl.ds(kc * _LN, _LN)]
                for l in range(_LN):
                    nkb = _vbcast(vk, l)
                    cnt = cnt + jnp.where(nkb <= njv, one, zero)
                return cnt

            cnt = lax.fori_loop(0, kb, before, zero)

            def middle(k, cnt):
                nkb = jnp.full((_LN,), noise_v[pl.ds(k, _LN)][0])
                lt = nkb < njv
                tie = (nkb == njv) & (k < jidx)
                return cnt + jnp.where(lt | tie, one, zero)

            cnt = lax.fori_loop(j0, j0 + _LN, middle, cnt)

            def after(kc, cnt):
                vk = noise_v[pl.ds(kc * _LN, _LN)]
                for l in range(_LN):
                    nkb = _vbcast(vk, l)
                    cnt = cnt + jnp.where(nkb < njv, one, zero)
                return cnt

            cnt = lax.fori_loop(kb + 1, nkc, after, cnt)

            mask_v[pl.ds(jc * _LN, _LN)] = jnp.where(
                cnt < float(num_keep), one, zero)

        pltpu.sync_copy(mask_v,
                        mask_hbm.at[pl.ds(b * _NP + part * _JPW, _JPW)])


def _sc_mask(noise):
    b = noise.shape[0]
    mesh = plsc.VectorSubcoreMesh(core_axis_name="c", subcore_axis_name="s")
    flat = pl.kernel(
        _sc_mask_kernel,
        mesh=mesh,
        out_type=jax.ShapeDtypeStruct((b * _NP,), jnp.float32),
        scratch_types=[
            pltpu.VMEM((_NP + _LN,), jnp.float32),
            pltpu.VMEM((_JPW,), jnp.float32),
        ],
    )(noise.reshape(b * _NP))
    return flat.reshape(b, _NP)


def _tc_multiply_kernel(mask_ref, x_ref, out_ref, spat_ref):
    nc = pl.program_id(1)
    hp = 224 // _P

    @pl.when(nc == 0)
    def _expand_mask():
        m = mask_ref[0]
        a_s = lax.broadcasted_iota(jnp.int32, (_ROWS, _NP), 0)
        a_p = lax.broadcasted_iota(jnp.int32, (_ROWS, _NP), 1)
        a_sel = ((a_p // hp) == (a_s // _P)).astype(jnp.float32)
        b_p = lax.broadcasted_iota(jnp.int32, (_NP, _LANES), 0)
        b_l = lax.broadcasted_iota(jnp.int32, (_NP, _LANES), 1)
        b_sel = ((b_p % hp) == (b_l // _P)).astype(jnp.float32)
        spat_ref[...] = jnp.dot(a_sel, m * b_sel,
                                preferred_element_type=jnp.float32)

    out_ref[...] = x_ref[...] * spat_ref[...][None, None, :, :]


def kernel(x, noise):
    b, c, h_full, w_full = x.shape
    num_patches = noise.shape[1]
    nc = c // _CC

    mask = _sc_mask(noise)
    mask3 = mask.reshape(b, num_patches, 1)

    x_img = pl.pallas_call(
        _tc_multiply_kernel,
        grid=(b, nc),
        in_specs=[
            pl.BlockSpec((1, num_patches, 1), lambda i, j: (i, 0, 0)),
            pl.BlockSpec((1, _CC, _ROWS, _LANES), lambda i, j: (i, j, 0, 0)),
        ],
        out_specs=pl.BlockSpec((1, _CC, _ROWS, _LANES),
                               lambda i, j: (i, j, 0, 0)),
        out_shape=jax.ShapeDtypeStruct((b, c, _ROWS, _LANES), x.dtype),
        scratch_shapes=[pltpu.VMEM((_ROWS, _LANES), jnp.float32)],
        compiler_params=pltpu.CompilerParams(
            dimension_semantics=("parallel", "arbitrary"),
        ),
    )(mask3, x)

    return (x_img, mask)


# speedup vs baseline: 1.0599x; 1.0599x over previous
import jax
import jax.numpy as jnp
from jax import lax
from jax.experimental import pallas as pl
from jax.experimental.pallas import tpu as pltpu
from jax.experimental.pallas import tpu_sc as plsc

_P = 8
_MASK_RATIO = 0.75
_CC = 64
_ROWS, _LANES = 224, 224
_NP = 784
_JPW = 112
_LN = 16
_NPAD = 1024


_GDN = lax.GatherDimensionNumbers(
    offset_dims=(), collapsed_slice_dims=(0,), start_index_map=(0,))


def _vbcast(v, l):
    idx = jnp.full((_LN, 1), l, jnp.int32)
    return lax.gather(v, idx, _GDN, slice_sizes=(1,),
                      mode=lax.GatherScatterMode.PROMISE_IN_BOUNDS)


def _sc_mask_kernel(noise_hbm, mask_hbm, noise_v, mask_v):
    b_total = noise_hbm.shape[0]
    num_keep = int(_NP * (1.0 - _MASK_RATIO))
    nworkers = b_total * (_NP // _JPW)

    wid = lax.axis_index("s") * 2 + lax.axis_index("c")

    @pl.when(wid < nworkers)
    def _():
        b = wid // (_NP // _JPW)
        part = wid % (_NP // _JPW)
        pltpu.sync_copy(noise_hbm, noise_v)

        lane = lax.broadcasted_iota(jnp.int32, (_LN,), 0)
        one = jnp.ones((_LN,), jnp.float32)
        zero = jnp.zeros((_LN,), jnp.float32)
        nkc = _NP // _LN
        nacc = 4

        for jc in range(_JPW // _LN):
            j0 = part * _JPW + jc * _LN
            njv = noise_v[b, pl.ds(j0, _LN)]
            kb = part * (_JPW // _LN) + jc

            def before(kc, cnts):
                vk = noise_v[b, pl.ds(kc * _LN, _LN)]
                out = list(cnts)
                for l in range(_LN):
                    nkb = _vbcast(vk, l)
                    out[l % nacc] = out[l % nacc] + jnp.where(
                        nkb <= njv, one, zero)
                return tuple(out)

            cnts = lax.fori_loop(0, kb, before, (zero,) * nacc)

            out = list(cnts)
            for l in range(_LN):
                nkb = _vbcast(njv, l)
                tl = lane > l
                hit = (nkb < njv) | ((nkb == njv) & tl)
                out[l % nacc] = out[l % nacc] + jnp.where(hit, one, zero)
            cnts = tuple(out)

            def after(kc, cnts):
                vk = noise_v[b, pl.ds(kc * _LN, _LN)]
                out = list(cnts)
                for l in range(_LN):
                    nkb = _vbcast(vk, l)
                    out[l % nacc] = out[l % nacc] + jnp.where(
                        nkb < njv, one, zero)
                return tuple(out)

            cnts = lax.fori_loop(kb + 1, nkc, after, cnts)

            rank = cnts[0] + cnts[1] + cnts[2] + cnts[3]
            mask_v[pl.ds(jc * _LN, _LN)] = jnp.where(
                rank < float(num_keep), one, zero)

        pltpu.sync_copy(mask_v,
                        mask_hbm.at[pl.ds(b * _NPAD + part * _JPW, _JPW)])


def _sc_mask(noise):
    b = noise.shape[0]
    mesh = plsc.VectorSubcoreMesh(core_axis_name="c", subcore_axis_name="s")
    return pl.kernel(
        _sc_mask_kernel,
        mesh=mesh,
        out_type=jax.ShapeDtypeStruct((b * _NPAD,), jnp.float32),
        scratch_types=[
            pltpu.VMEM((b, _NP), jnp.float32),
            pltpu.VMEM((_JPW,), jnp.float32),
        ],
    )(noise)


def _tc_multiply_kernel(mask_ref, x_ref, out_ref, mask2_ref, spat_ref):
    bi = pl.program_id(0)
    nc = pl.program_id(1)
    hp = 224 // _P

    @pl.when(nc == 0)
    def _expand_mask():
        m = mask_ref[pl.ds(bi * _NPAD, _NP)]
        mask2_ref[pl.ds(bi, 1), :] = m[None, :]
        a_s = lax.broadcasted_iota(jnp.int32, (_ROWS, _NP), 0)
        a_p = lax.broadcasted_iota(jnp.int32, (_ROWS, _NP), 1)
        a_sel = ((a_p // hp) == (a_s // _P)).astype(jnp.float32)
        b_p = lax.broadcasted_iota(jnp.int32, (_NP, _LANES), 0)
        b_l = lax.broadcasted_iota(jnp.int32, (_NP, _LANES), 1)
        b_sel = ((b_p % hp) == (b_l // _P)).astype(jnp.float32)
        spat_ref[...] = jnp.dot(a_sel * m[None, :], b_sel,
                                preferred_element_type=jnp.float32)

    out_ref[...] = x_ref[...] * spat_ref[...][None, None, :, :]


def kernel(x, noise):
    b, c, h_full, w_full = x.shape
    num_patches = noise.shape[1]
    nc = c // _CC

    mask_flat = _sc_mask(noise)

    x_img, mask2 = pl.pallas_call(
        _tc_multiply_kernel,
        grid=(b, nc),
        in_specs=[
            pl.BlockSpec((b * _NPAD,), lambda i, j: (0,)),
            pl.BlockSpec((1, _CC, _ROWS, _LANES), lambda i, j: (i, j, 0, 0)),
        ],
        out_specs=[
            pl.BlockSpec((1, _CC, _ROWS, _LANES), lambda i, j: (i, j, 0, 0)),
            pl.BlockSpec((b, num_patches), lambda i, j: (0, 0)),
        ],
        out_shape=[
            jax.ShapeDtypeStruct((b, c, _ROWS, _LANES), x.dtype),
            jax.ShapeDtypeStruct((b, num_patches), jnp.float32),
        ],
        scratch_shapes=[pltpu.VMEM((_ROWS, _LANES), jnp.float32)],
        compiler_params=pltpu.CompilerParams(
            dimension_semantics=("arbitrary", "arbitrary"),
        ),
    )(mask_flat, x)

    return (x_img, mask2)


# speedup vs baseline: 1.0745x; 1.0138x over previous
import jax
import jax.numpy as jnp
from jax import lax
from jax.experimental import pallas as pl
from jax.experimental.pallas import tpu as pltpu
from jax.experimental.pallas import tpu_sc as plsc

_P = 8
_MASK_RATIO = 0.75
_CC = 64
_ROWS, _LANES = 224, 224
_NP = 784
_JPW = 112
_LN = 16
_NPAD = 1024


_GDN = lax.GatherDimensionNumbers(
    offset_dims=(), collapsed_slice_dims=(0,), start_index_map=(0,))


def _vbcast(v, l):
    idx = jnp.full((_LN, 1), l, jnp.int32)
    return lax.gather(v, idx, _GDN, slice_sizes=(1,),
                      mode=lax.GatherScatterMode.PROMISE_IN_BOUNDS)


def _sc_mask_kernel(noise_hbm, mask_hbm, noise_v, mask_v):
    b_total = noise_hbm.shape[0]
    num_keep = int(_NP * (1.0 - _MASK_RATIO))
    nworkers = b_total * (_NP // _JPW)

    wid = lax.axis_index("s") * 2 + lax.axis_index("c")

    @pl.when(wid < nworkers)
    def _():
        b = wid // (_NP // _JPW)
        part = wid % (_NP // _JPW)
        pltpu.sync_copy(noise_hbm, noise_v)

        lane = lax.broadcasted_iota(jnp.int32, (_LN,), 0)
        one = jnp.ones((_LN,), jnp.float32)
        zero = jnp.zeros((_LN,), jnp.float32)
        nkc = _NP // _LN
        nacc = 4

        def per_chunk(jc, _):
            j0 = part * _JPW + jc * _LN
            njv = noise_v[b, pl.ds(j0, _LN)]
            kb = part * (_JPW // _LN) + jc

            def before(kc, cnts):
                vk = noise_v[b, pl.ds(kc * _LN, _LN)]
                out = list(cnts)
                for l in range(_LN):
                    nkb = _vbcast(vk, l)
                    out[l % nacc] = out[l % nacc] + jnp.where(
                        nkb <= njv, one, zero)
                return tuple(out)

            cnts = lax.fori_loop(0, kb, before, (zero,) * nacc)

            out = list(cnts)
            for l in range(_LN):
                nkb = _vbcast(njv, l)
                tl = lane > l
                hit = (nkb < njv) | ((nkb == njv) & tl)
                out[l % nacc] = out[l % nacc] + jnp.where(hit, one, zero)
            cnts = tuple(out)

            def after(kc, cnts):
                vk = noise_v[b, pl.ds(kc * _LN, _LN)]
                out = list(cnts)
                for l in range(_LN):
                    nkb = _vbcast(vk, l)
                    out[l % nacc] = out[l % nacc] + jnp.where(
                        nkb < njv, one, zero)
                return tuple(out)

            cnts = lax.fori_loop(kb + 1, nkc, after, cnts)

            rank = cnts[0] + cnts[1] + cnts[2] + cnts[3]
            mask_v[pl.ds(jc * _LN, _LN)] = jnp.where(
                rank < float(num_keep), one, zero)
            return 0

        lax.fori_loop(0, _JPW // _LN, per_chunk, 0)

        pltpu.sync_copy(mask_v,
                        mask_hbm.at[pl.ds(b * _NPAD + part * _JPW, _JPW)])


def _sc_mask(noise):
    b = noise.shape[0]
    mesh = plsc.VectorSubcoreMesh(core_axis_name="c", subcore_axis_name="s")
    return pl.kernel(
        _sc_mask_kernel,
        mesh=mesh,
        out_type=jax.ShapeDtypeStruct((b * _NPAD,), jnp.float32),
        scratch_types=[
            pltpu.VMEM((b, _NP), jnp.float32),
            pltpu.VMEM((_JPW,), jnp.float32),
        ],
    )(noise)


def _tc_multiply_kernel(mask_ref, x_ref, out_ref, mask2_ref, spat_ref):
    bi = pl.program_id(0)
    nc = pl.program_id(1)
    hp = 224 // _P

    @pl.when(nc == 0)
    def _expand_mask():
        m = mask_ref[pl.ds(bi * _NPAD, _NP)]
        mask2_ref[pl.ds(bi, 1), :] = m[None, :]
        a_s = lax.broadcasted_iota(jnp.int32, (_ROWS, _NP), 0)
        a_p = lax.broadcasted_iota(jnp.int32, (_ROWS, _NP), 1)
        a_sel = ((a_p // hp) == (a_s // _P)).astype(jnp.float32)
        b_p = lax.broadcasted_iota(jnp.int32, (_NP, _LANES), 0)
        b_l = lax.broadcasted_iota(jnp.int32, (_NP, _LANES), 1)
        b_sel = ((b_p % hp) == (b_l // _P)).astype(jnp.float32)
        spat_ref[...] = jnp.dot(a_sel * m[None, :], b_sel,
                                preferred_element_type=jnp.float32)

    out_ref[...] = x_ref[...] * spat_ref[...][None, None, :, :]


def kernel(x, noise):
    b, c, h_full, w_full = x.shape
    num_patches = noise.shape[1]
    nc = c // _CC

    mask_flat = _sc_mask(noise)

    x_img, mask2 = pl.pallas_call(
        _tc_multiply_kernel,
        grid=(b, nc),
        in_specs=[
            pl.BlockSpec((b * _NPAD,), lambda i, j: (0,)),
            pl.BlockSpec((1, _CC, _ROWS, _LANES), lambda i, j: (i, j, 0, 0)),
        ],
        out_specs=[
            pl.BlockSpec((1, _CC, _ROWS, _LANES), lambda i, j: (i, j, 0, 0)),
            pl.BlockSpec((b, num_patches), lambda i, j: (0, 0)),
        ],
        out_shape=[
            jax.ShapeDtypeStruct((b, c, _ROWS, _LANES), x.dtype),
            jax.ShapeDtypeStruct((b, num_patches), jnp.float32),
        ],
        scratch_shapes=[pltpu.VMEM((_ROWS, _LANES), jnp.float32)],
        compiler_params=pltpu.CompilerParams(
            dimension_semantics=("arbitrary", "arbitrary"),
        ),
    )(mask_flat, x)

    return (x_img, mask2)


# speedup vs baseline: 1.1082x; 1.0314x over previous
import jax
import jax.numpy as jnp
from jax import lax
from jax.experimental import pallas as pl
from jax.experimental.pallas import tpu as pltpu
from jax.experimental.pallas import tpu_sc as plsc

_P = 8
_MASK_RATIO = 0.75
_CC = 64
_ROWS, _LANES = 224, 224
_NP = 784
_JPW = 112
_LN = 16
_NPAD = 1024


_GDN = lax.GatherDimensionNumbers(
    offset_dims=(), collapsed_slice_dims=(0,), start_index_map=(0,))


def _vbcast(v, l):
    idx = jnp.full((_LN, 1), l, jnp.int32)
    return lax.gather(v, idx, _GDN, slice_sizes=(1,),
                      mode=lax.GatherScatterMode.PROMISE_IN_BOUNDS)


def _sc_mask_kernel(noise_hbm, mask_hbm, noise_v, mask_v):
    b_total = noise_hbm.shape[0]
    num_keep = int(_NP * (1.0 - _MASK_RATIO))
    nworkers = b_total * (_NP // _JPW)

    wid = lax.axis_index("s") * 2 + lax.axis_index("c")

    @pl.when(wid < nworkers)
    def _():
        b = wid // (_NP // _JPW)
        part = wid % (_NP // _JPW)
        pltpu.sync_copy(noise_hbm, noise_v)

        lane = lax.broadcasted_iota(jnp.int32, (_LN,), 0)
        one = jnp.ones((_LN,), jnp.float32)
        zero = jnp.zeros((_LN,), jnp.float32)
        nkc = _NP // _LN
        nacc = 4

        def per_chunk(jc, _):
            j0 = part * _JPW + jc * _LN
            njv = noise_v[b, pl.ds(j0, _LN)]
            kb = part * (_JPW // _LN) + jc

            def before(kc, cnts):
                vk = noise_v[b, pl.ds(kc * _LN, _LN)]
                out = list(cnts)
                for l in range(_LN):
                    nkb = _vbcast(vk, l)
                    out[l % nacc] = out[l % nacc] + jnp.where(
                        nkb <= njv, one, zero)
                return tuple(out)

            cnts = lax.fori_loop(0, kb, before, (zero,) * nacc)

            out = list(cnts)
            for l in range(_LN):
                nkb = _vbcast(njv, l)
                tl = lane > l
                hit = (nkb < njv) | ((nkb == njv) & tl)
                out[l % nacc] = out[l % nacc] + jnp.where(hit, one, zero)
            cnts = tuple(out)

            def after(kc, cnts):
                vk = noise_v[b, pl.ds(kc * _LN, _LN)]
                out = list(cnts)
                for l in range(_LN):
                    nkb = _vbcast(vk, l)
                    out[l % nacc] = out[l % nacc] + jnp.where(
                        nkb < njv, one, zero)
                return tuple(out)

            cnts = lax.fori_loop(kb + 1, nkc, after, cnts)

            rank = cnts[0] + cnts[1] + cnts[2] + cnts[3]
            mask_v[pl.ds(jc * _LN, _LN)] = jnp.where(
                rank < float(num_keep), one, zero)
            return 0

        lax.fori_loop(0, _JPW // _LN, per_chunk, 0)

        pltpu.sync_copy(mask_v,
                        mask_hbm.at[pl.ds(b * _NP + part * _JPW, _JPW)])


def _sc_mask(noise):
    b = noise.shape[0]
    mesh = plsc.VectorSubcoreMesh(core_axis_name="c", subcore_axis_name="s")
    return pl.kernel(
        _sc_mask_kernel,
        mesh=mesh,
        out_type=jax.ShapeDtypeStruct((b * _NP,), jnp.float32),
        scratch_types=[
            pltpu.VMEM((b, _NP), jnp.float32),
            pltpu.VMEM((_JPW,), jnp.float32),
        ],
    )(noise)


def _tc_multiply_kernel(noise_j_ref, noise_k_ref, x_ref, out_ref, spat_ref):
    nc = pl.program_id(1)
    hp = 224 // _P
    num_keep = int(_NP * (1.0 - _MASK_RATIO))

    @pl.when(nc == 0)
    def _compute_mask():
        nj = noise_j_ref[0]
        nk = noise_k_ref[0]
        j_idx = lax.broadcasted_iota(jnp.int32, (_NP, _NP), 0)
        k_idx = lax.broadcasted_iota(jnp.int32, (_NP, _NP), 1)
        lt = nk < nj
        tie = (nk == nj) & (k_idx < j_idx)
        rank = jnp.sum((lt | tie).astype(jnp.float32), axis=1, keepdims=True)
        m = (rank < num_keep).astype(jnp.float32)

        a_s = lax.broadcasted_iota(jnp.int32, (_ROWS, _NP), 0)
        a_p = lax.broadcasted_iota(jnp.int32, (_ROWS, _NP), 1)
        a_sel = ((a_p // hp) == (a_s // _P)).astype(jnp.float32)
        b_p = lax.broadcasted_iota(jnp.int32, (_NP, _LANES), 0)
        b_l = lax.broadcasted_iota(jnp.int32, (_NP, _LANES), 1)
        b_sel = ((b_p % hp) == (b_l // _P)).astype(jnp.float32)
        spat_ref[...] = jnp.dot(a_sel, m * b_sel,
                                preferred_element_type=jnp.float32)

    out_ref[...] = x_ref[...] * spat_ref[...][None, None, :, :]


def kernel(x, noise):
    b, c, h_full, w_full = x.shape
    num_patches = noise.shape[1]
    nc = c // _CC

    mask_flat = _sc_mask(noise)

    noise_j = noise.reshape(b, num_patches, 1)
    noise_k = noise.reshape(b, 1, num_patches)

    x_img = pl.pallas_call(
        _tc_multiply_kernel,
        grid=(b, nc),
        in_specs=[
            pl.BlockSpec((1, num_patches, 1), lambda i, j: (i, 0, 0)),
            pl.BlockSpec((1, 1, num_patches), lambda i, j: (i, 0, 0)),
            pl.BlockSpec((1, _CC, _ROWS, _LANES), lambda i, j: (i, j, 0, 0)),
        ],
        out_specs=pl.BlockSpec((1, _CC, _ROWS, _LANES),
                               lambda i, j: (i, j, 0, 0)),
        out_shape=jax.ShapeDtypeStruct((b, c, _ROWS, _LANES), x.dtype),
        scratch_shapes=[pltpu.VMEM((_ROWS, _LANES), jnp.float32)],
        compiler_params=pltpu.CompilerParams(
            dimension_semantics=("arbitrary", "arbitrary"),
        ),
    )(noise_j, noise_k, x)

    return (x_img, mask_flat.reshape(b, num_patches))
